# baseline (speedup 1.0000x reference)
import jax
import jax.numpy as jnp
from jax import lax
from jax.experimental import pallas as pl
from jax.experimental.pallas import tpu as pltpu
from jax.experimental.pallas import tpu_sc as plsc

VOCAB = 100000
HIDDEN = 768
MAX_POS = 4096
EPS = 1e-12
B, S = 4, 4096
N = B * S

NC, NS, L = 2, 16, 16
NW = NC * NS
TPW = N // NW
C = 64
NCHUNK = TPW // C
NV = HIDDEN // L


def _body(tid_hbm, tt_hbm, word_hbm, pos_hbm, type_hbm, gamma_hbm, beta_hbm,
          out_hbm, idx_v, tt_v, prow, wrow, type_v, gamma_v, beta_v, sem):
    cid = lax.axis_index("c")
    sid = lax.axis_index("s")
    wid = sid * NC + cid
    base = wid * TPW

    pltpu.sync_copy(type_hbm, type_v)
    pltpu.sync_copy(gamma_hbm, gamma_v)
    pltpu.sync_copy(beta_hbm, beta_v)

    def chunk_body(c, carry):
        g = base + c * C
        p0 = lax.rem(g, S)
        pltpu.sync_copy(tid_hbm.at[pl.ds(g, C)], idx_v)
        pltpu.sync_copy(tt_hbm.at[pl.ds(g, C)], tt_v.at[pl.ds(0, C)])
        pltpu.sync_copy(pos_hbm.at[pl.ds(p0, C)], prow)
        pltpu.async_copy(word_hbm.at[idx_v], wrow, sem).wait()

        def tok_body(t, tcarry):
            ttf = tt_v[pl.ds(t, L)][0].astype(jnp.float32)
            acc_s = jnp.zeros((L,), jnp.float32)
            acc_q = jnp.zeros((L,), jnp.float32)
            for j in range(NV):
                sl = pl.ds(j * L, L)
                row = (wrow[t, sl] + prow[t, sl] + type_v[0, sl]
                       + ttf * (type_v[1, sl] - type_v[0, sl]))
                wrow[t, sl] = row
                acc_s = acc_s + row
                acc_q = acc_q + row * row
            mean = jnp.sum(acc_s) * (1.0 / HIDDEN)
            var = jnp.sum(acc_q) * (1.0 / HIDDEN) - mean * mean
            x = jnp.full((L,), var + EPS, jnp.float32)
            seed = 0x5F3759DF - lax.shift_right_logical(
                plsc.bitcast(x, jnp.int32), 1)
            y = plsc.bitcast(seed, jnp.float32)
            hx = x * 0.5
            y = y * (1.5 - hx * y * y)
            y = y * (1.5 - hx * y * y)
            y = y * (1.5 - hx * y * y)
            meanv = jnp.full((L,), mean, jnp.float32)
            for j in range(NV):
                sl = pl.ds(j * L, L)
                wrow[t, sl] = ((wrow[t, sl] - meanv) * y * gamma_v[sl]
                               + beta_v[sl])
            return tcarry

        lax.fori_loop(0, C, tok_body, 0)
        pltpu.sync_copy(wrow, out_hbm.at[pl.ds(g, C)])
        return carry

    lax.fori_loop(0, NCHUNK, chunk_body, 0)


def kernel(token_ids, token_type_ids, word_emb, pos_emb, type_emb, gamma, beta):
    tid = token_ids.reshape(N).astype(jnp.int32)
    tt = token_type_ids.reshape(N).astype(jnp.int32)
    mesh = plsc.VectorSubcoreMesh(core_axis_name="c", subcore_axis_name="s",
                                  num_cores=NC, num_subcores=NS)
    out = pl.kernel(
        _body,
        out_type=jax.ShapeDtypeStruct((N, HIDDEN), jnp.float32),
        mesh=mesh,
        compiler_params=pltpu.CompilerParams(needs_layout_passes=False),
        scratch_types=[
            pltpu.VMEM((C,), jnp.int32),
            pltpu.VMEM((C + L,), jnp.int32),
            pltpu.VMEM((C, HIDDEN), jnp.float32),
            pltpu.VMEM((C, HIDDEN), jnp.float32),
            pltpu.VMEM((2, HIDDEN), jnp.float32),
            pltpu.VMEM((HIDDEN,), jnp.float32),
            pltpu.VMEM((HIDDEN,), jnp.float32),
            pltpu.SemaphoreType.DMA,
        ],
    )(tid, tt, word_emb, pos_emb, type_emb, gamma, beta)
    return out.reshape(B, S, HIDDEN)
